# Initial kernel scaffold; baseline (speedup 1.0000x reference)
import jax
import jax.numpy as jnp
from jax.experimental import pallas as pl


def kernel(weight):
    raise NotImplementedError("write your pallas kernel here")


# speedup vs baseline: 45.3522x; 45.3522x over previous
import functools

import jax
import jax.numpy as jnp
from jax import lax
from jax.experimental import pallas as pl
from jax.experimental.pallas import tpu as pltpu
from jax.experimental.pallas import tpu_sc as plsc

NC = 2
NS = 16
LANES = 16
NW = NC * NS

ROWS, COLS = 4096, 16384
TOTAL = ROWS * COLS
PER_W = TOTAL // NW
CHUNK = 32768
N_CHUNKS = PER_W // CHUNK


def _mask_body(buf, i):
    x = buf[pl.ds(i, 16)]
    a = jnp.abs(x)
    lane = lax.iota(jnp.int32, 16)
    rank = jnp.zeros(16, jnp.int32)
    for s in range(1, 8):
        perm = (lane & 8) | ((lane + s) & 7)
        aj = jnp.take_along_axis(a, perm, axis=0)
        tiem = ((lane & 7) + s) >= 8
        beat = (aj > a) | ((aj == a) & tiem)
        rank = rank + jnp.where(beat, 1, 0)
    keep = rank < 4
    buf[pl.ds(i, 16)] = jnp.where(keep, x, 0.0)


def _make_pruner():
    mesh = plsc.VectorSubcoreMesh(core_axis_name="c", subcore_axis_name="s")

    @functools.partial(
        pl.kernel,
        mesh=mesh,
        out_type=jax.ShapeDtypeStruct((TOTAL,), jnp.float32),
        scratch_types=[pltpu.VMEM((CHUNK,), jnp.float32)],
    )
    def pruner(w_hbm, out_hbm, buf):
        wid = lax.axis_index("s") * NC + lax.axis_index("c")
        base_w = wid * PER_W

        def chunk_body(c, carry):
            base = base_w + c * CHUNK
            pltpu.sync_copy(w_hbm.at[pl.ds(base, CHUNK)], buf)

            @plsc.parallel_loop(0, CHUNK, 16, unroll=8)
            def _(i):
                _mask_body(buf, i)

            pltpu.sync_copy(buf, out_hbm.at[pl.ds(base, CHUNK)])
            return carry

        lax.fori_loop(0, N_CHUNKS, chunk_body, 0)

    return pruner


_pruner = _make_pruner()


def kernel(weight):
    flat = weight.reshape(TOTAL)
    out = _pruner(flat)
    return out.reshape(ROWS, COLS)
